# baseline (speedup 1.0000x reference)
import functools

import jax
import jax.numpy as jnp
from jax import lax
from jax.experimental import pallas as pl
from jax.experimental.pallas import tpu as pltpu
from jax.experimental.pallas import tpu_sc as plsc

NC = 2
NS = 16
NW = NC * NS


def _mesh():
    return plsc.VectorSubcoreMesh(core_axis_name="c", subcore_axis_name="s")


def _sc_gather_rows(table, idx_flat, ch=128):
    n = idx_flat.shape[0]
    e = table.shape[1]
    per_w = n // NW
    n_ch = per_w // ch

    @functools.partial(
        pl.kernel,
        out_type=jax.ShapeDtypeStruct((n, e), jnp.float32),
        mesh=_mesh(),
        scratch_types=[
            pltpu.VMEM((per_w,), jnp.int32),
            pltpu.VMEM((ch,), jnp.int32),
            pltpu.VMEM((ch,), jnp.int32),
            pltpu.VMEM((ch, e), jnp.float32),
            pltpu.VMEM((ch, e), jnp.float32),
            pltpu.SemaphoreType.DMA,
            pltpu.SemaphoreType.DMA,
            pltpu.SemaphoreType.DMA,
            pltpu.SemaphoreType.DMA,
        ],
    )
    def k(table_hbm, idx_hbm, out_hbm, ids_v, idx0, idx1, buf0, buf1,
          sg0, sg1, sw0, sw1):
        wid = lax.axis_index("s") * NC + lax.axis_index("c")
        base = wid * per_w
        idxs = (idx0, idx1)
        bufs = (buf0, buf1)
        sgs = (sg0, sg1)
        sws = (sw0, sw1)
        pltpu.sync_copy(idx_hbm.at[pl.ds(base, per_w)], ids_v)

        def start_gather(r, ci):
            for kk in range(ch // 16):
                sl = pl.ds(kk * 16, 16)
                idxs[r][sl] = ids_v[pl.ds(ci * ch + kk * 16, 16)]
            pltpu.async_copy(table_hbm.at[idxs[r]], bufs[r], sgs[r])

        def finish_and_write(r, ci):
            pltpu.make_async_copy(table_hbm.at[idxs[r]], bufs[r],
                                  sgs[r]).wait()
            pltpu.async_copy(bufs[r], out_hbm.at[pl.ds(base + ci * ch, ch)],
                             sws[r])

        def wait_write(r):
            pltpu.make_async_copy(bufs[r], out_hbm.at[pl.ds(0, ch)],
                                  sws[r]).wait()

        def body2(o, carry):
            for r in (0, 1):
                ci = 2 * o + r

                @pl.when(ci >= 2)
                def _():
                    wait_write(r)

                start_gather(r, ci)

                @pl.when(ci >= 1)
                def _():
                    finish_and_write(1 - r, ci - 1)

            return carry

        lax.fori_loop(0, n_ch // 2, body2, 0)
        finish_and_write((n_ch - 1) % 2, n_ch - 1)
        wait_write(0)
        wait_write(1)

    return k(table, idx_flat)


def _tc_node_linear(repr3, head_ids, tail_ids, rel_ids,
                    rel_table, ws, wn, wr):
    bsz, m, e = repr3.shape
    mt = head_ids.shape[1]
    r = rel_table.shape[0]
    head3 = head_ids.reshape(bsz, 1, mt)
    tail3 = tail_ids.reshape(bsz, 1, mt)
    rel3 = rel_ids.reshape(bsz, 1, mt)
    f32 = jnp.float32
    bf16 = jnp.bfloat16

    def body(repr_ref, head_ref, tail_ref, relid_ref,
             rel_ref, ws_ref, wn_ref, wr_ref, out_ref):
        x16 = repr_ref[0].astype(bf16)
        iota_g = lax.broadcasted_iota(jnp.int32, (m, mt), 0)
        ah = (iota_g == head_ref[0]).astype(bf16)
        at = (iota_g == tail_ref[0]).astype(bf16)
        dn_min = (((1,), (1,)), ((), ()))
        badj = lax.dot_general(at, ah, dn_min, preferred_element_type=f32)
        badj += lax.dot_general(ah, at, dn_min, preferred_element_type=f32)
        deg = jnp.sum(badj, axis=1, keepdims=True)
        upd = lax.dot_general(badj.astype(bf16), x16,
                              (((1,), (0,)), ((), ())),
                              preferred_element_type=f32)
        inv = 1.0 / jnp.maximum(deg, 1.0)
        acc = lax.dot_general(x16, ws_ref[...], dn_min,
                              preferred_element_type=f32)
        acc += lax.dot_general((upd * inv).astype(bf16), wn_ref[...], dn_min,
                               preferred_element_type=f32)
        node = jnp.maximum(acc, 0.0).astype(bf16)

        rel2 = lax.dot_general(rel_ref[...], wr_ref[...], dn_min,
                               preferred_element_type=f32).astype(bf16)

        dn_sub = (((0,), (0,)), ((), ()))
        iota_r = lax.broadcasted_iota(jnp.int32, (r, mt), 0)
        ar = (iota_r == relid_ref[0]).astype(bf16)
        out_ref[0, :, 0:e] = lax.dot_general(ah, node, dn_sub,
                                             preferred_element_type=f32)
        out_ref[0, :, e:2 * e] = lax.dot_general(ar, rel2, dn_sub,
                                                 preferred_element_type=f32)
        out_ref[0, :, 2 * e:3 * e] = lax.dot_general(at, node, dn_sub,
                                                     preferred_element_type=f32)

    return pl.pallas_call(
        body,
        grid=(bsz,),
        in_specs=[
            pl.BlockSpec((1, m, e), lambda b: (b, 0, 0)),
            pl.BlockSpec((1, 1, mt), lambda b: (b, 0, 0)),
            pl.BlockSpec((1, 1, mt), lambda b: (b, 0, 0)),
            pl.BlockSpec((1, 1, mt), lambda b: (b, 0, 0)),
            pl.BlockSpec((r, e), lambda b: (0, 0)),
            pl.BlockSpec((e, e), lambda b: (0, 0)),
            pl.BlockSpec((e, e), lambda b: (0, 0)),
            pl.BlockSpec((e, e), lambda b: (0, 0)),
        ],
        out_specs=pl.BlockSpec((1, mt, 3 * e), lambda b: (b, 0, 0)),
        out_shape=jax.ShapeDtypeStruct((bsz, mt, 3 * e), f32),
    )(repr3, head3, tail3, rel3, rel_table, ws, wn, wr)


def kernel(concept_ids, relations, head_ids, tail_ids, concept_table,
           rel_table, W_s, W_n, W_r):
    bsz, m = concept_ids.shape
    mt = head_ids.shape[1]
    e = concept_table.shape[1]

    cids = concept_ids.astype(jnp.int32)
    rels = relations.astype(jnp.int32)
    hids = head_ids.astype(jnp.int32)
    tids = tail_ids.astype(jnp.int32)
    ws = W_s[-1].astype(jnp.float32)
    wn = W_n[-1].astype(jnp.float32)
    wr = W_r[-1].astype(jnp.float32)

    repr_flat = _sc_gather_rows(concept_table, cids.reshape(-1))

    repr3 = repr_flat.reshape(bsz, m, e)
    return _tc_node_linear(repr3, hids, tids, rels,
                           rel_table.astype(jnp.float32),
                           ws.astype(jnp.bfloat16), wn.astype(jnp.bfloat16),
                           wr)
